# Initial kernel scaffold; baseline (speedup 1.0000x reference)
import jax
import jax.numpy as jnp
from jax.experimental import pallas as pl


def kernel(x, edge_index, edge_attr, batch, node_W, node_b, lin1_W, lin1_b, lin2_W, lin3_W, lin3_b, pred_W, pred_b):
    raise NotImplementedError("write your pallas kernel here")


# speedup vs baseline: 5.2164x; 5.2164x over previous
import functools

import jax
import jax.numpy as jnp
from jax import lax
from jax.experimental import pallas as pl
from jax.experimental.pallas import tpu as pltpu
from jax.experimental.pallas import tpu_sc as plsc

_N = 10000
_E = 320000
_H = 128
_NG = 128
_NCLS = 10
_NSC = 2
_NSUB = 16
_NW = _NSC * _NSUB
_EPW = _E // _NW
_K = 80
_NCH = _EPW // _K
_RPS = _N // _NSUB
_R = 1000
_NB = _N // _R

_mesh = plsc.VectorSubcoreMesh(
    core_axis_name="c", subcore_axis_name="s",
    num_cores=_NSC, num_subcores=_NSUB)


def _edge_body(with_deg, src_hbm, dst_hbm, w_hbm, a_hbm, zrows_hbm, *rest):
    if with_deg:
        (zdeg_hbm, out_hbm, deg_out_hbm,
         src_v, dst_v, w_v, rows_v, acc_sh, deg_sh, deg_v, sem) = rest
    else:
        (out_hbm, src_v, dst_v, w_v, rows_v, acc_sh, sem) = rest

    cid = lax.axis_index("c")
    sid = lax.axis_index("s")
    wid = sid * _NSC + cid

    pltpu.sync_copy(zrows_hbm, acc_sh.at[pl.ds(sid * _RPS, _RPS)])
    if with_deg:
        @pl.when(sid == 0)
        def _():
            pltpu.sync_copy(zdeg_hbm, deg_v)
            pltpu.sync_copy(deg_v, deg_sh)
    plsc.subcore_barrier()

    ebase = wid * _EPW

    def chunk_body(ch, carry):
        off = ebase + ch * _K
        pltpu.sync_copy(src_hbm.at[pl.ds(off, _K)], src_v)
        pltpu.sync_copy(dst_hbm.at[pl.ds(off, _K)], dst_v)
        pltpu.sync_copy(w_hbm.at[pl.ds(off, _K)], w_v)
        pltpu.async_copy(a_hbm.at[src_v], rows_v, sem).wait()

        def scale_group(g, c2):
            w16 = w_v[pl.ds(g * 16, 16)]
            for j in range(16):
                wv = w16[j]
                e = g * 16 + j
                for k in range(_H // 16):
                    sl = pl.ds(k * 16, 16)
                    rows_v[e, sl] = rows_v[e, sl] * wv
            return c2
        lax.fori_loop(0, _K // 16, scale_group, 0)

        pltpu.sync_copy(rows_v, acc_sh.at[dst_v], add=True)
        if with_deg:
            pltpu.sync_copy(w_v, deg_sh.at[dst_v], add=True)
        return carry

    lax.fori_loop(0, _NCH, chunk_body, 0)
    plsc.subcore_barrier()

    @pl.when(sid < 2)
    def _():
        half = _N // 2
        pltpu.sync_copy(acc_sh.at[pl.ds(sid * half, half)],
                        out_hbm.at[pl.ds(cid * _N + sid * half, half)])
    if with_deg:
        @pl.when(sid == 0)
        def _():
            pltpu.sync_copy(deg_sh, deg_v)
            pltpu.sync_copy(deg_v, deg_out_hbm.at[pl.ds(cid * _N, _N)])


def _make_edge_pass(with_deg):
    out_type = [jax.ShapeDtypeStruct((_NSC * _N, _H), jnp.float32)]
    if with_deg:
        out_type.append(jax.ShapeDtypeStruct((_NSC * _N,), jnp.float32))
    scratch = [
        pltpu.VMEM((_K,), jnp.int32),
        pltpu.VMEM((_K,), jnp.int32),
        pltpu.VMEM((_K,), jnp.float32),
        pltpu.VMEM((_K, _H), jnp.float32),
        pltpu.VMEM_SHARED((_N, _H), jnp.float32),
    ]
    if with_deg:
        scratch.append(pltpu.VMEM_SHARED((_N,), jnp.float32))
        scratch.append(pltpu.VMEM((_N,), jnp.float32))
    scratch.append(pltpu.SemaphoreType.DMA)
    return pl.kernel(
        functools.partial(_edge_body, with_deg),
        out_type=out_type, mesh=_mesh, scratch_types=scratch)


_edge_pass_deg = _make_edge_pass(True)
_edge_pass = _make_edge_pass(False)


def _dot(a, b):
    return jnp.dot(a, b, preferred_element_type=jnp.float32)


def _dense0_body(x_ref, nw, nb, w1, b1, w2, w3, b3, a_ref, b_ref, c_ref):
    h = _dot(x_ref[...], nw[...]) + nb[...]
    a_ref[...] = _dot(h, w1[...]) + b1[...]
    b_ref[...] = _dot(h, w2[...])
    c_ref[...] = _dot(h, w3[...]) + b3[...]


def _combine_body(p0, p1, d0, d1, bp, cp, w1, b1, w2, w3, b3,
                  a_ref, b_ref, c_ref):
    deg = d0[0, 0] + d1[0, 0]
    h = jnp.maximum(p0[...] + p1[...] - deg * bp[...] + cp[...], 0.0)
    a_ref[...] = _dot(h, w1[...]) + b1[...]
    b_ref[...] = _dot(h, w2[...])
    c_ref[...] = _dot(h, w3[...]) + b3[...]


def _pool_body(p0, p1, d0, d1, bp, cp, batch_ref, pw, pb, out_ref,
               sums, cnt):
    i = pl.program_id(0)

    @pl.when(i == 0)
    def _():
        sums[...] = jnp.zeros_like(sums)
        cnt[...] = jnp.zeros_like(cnt)

    deg = d0[0, 0] + d1[0, 0]
    h = jnp.maximum(p0[...] + p1[...] - deg * bp[...] + cp[...], 0.0)
    brow = batch_ref[0]
    gids = lax.broadcasted_iota(jnp.int32, (_NG, _R), 0)
    onehot = (gids == brow).astype(jnp.float32)
    sums[...] += _dot(onehot, h)
    cnt[...] += _dot(onehot, jnp.ones((_R, _H), jnp.float32))

    @pl.when(i == _NB - 1)
    def _():
        hg = sums[...] / jnp.maximum(cnt[...], 1.0)
        out_ref[...] = _dot(hg, pw[...]) + pb[...]


_rowspec = pl.BlockSpec((_R, _H), lambda i: (i, 0))
_rowspec1 = pl.BlockSpec((_R, _H), lambda i: (i + _NB, 0))
_wspec = pl.BlockSpec((_H, _H), lambda i: (0, 0))
_bspec = pl.BlockSpec((1, _H), lambda i: (0, 0))
_d0spec = pl.BlockSpec((1, 1, _R, 1), lambda i: (0, i, 0, 0))
_d1spec = pl.BlockSpec((1, 1, _R, 1), lambda i: (1, i, 0, 0))

_dense0 = pl.pallas_call(
    _dense0_body,
    grid=(_NB,),
    in_specs=[_rowspec, _wspec, _bspec, _wspec, _bspec, _wspec, _wspec,
              _bspec],
    out_specs=[_rowspec, _rowspec, _rowspec],
    out_shape=[jax.ShapeDtypeStruct((_N, _H), jnp.float32)] * 3,
)

_combine_dense = pl.pallas_call(
    _combine_body,
    grid=(_NB,),
    in_specs=[_rowspec, _rowspec1, _d0spec, _d1spec, _rowspec, _rowspec,
              _wspec, _bspec, _wspec, _wspec, _bspec],
    out_specs=[_rowspec, _rowspec, _rowspec],
    out_shape=[jax.ShapeDtypeStruct((_N, _H), jnp.float32)] * 3,
)

_pool_head = pl.pallas_call(
    _pool_body,
    grid=(_NB,),
    in_specs=[_rowspec, _rowspec1, _d0spec, _d1spec, _rowspec, _rowspec,
              pl.BlockSpec((1, 1, _R), lambda i: (i, 0, 0)),
              pl.BlockSpec((_H, _NCLS), lambda i: (0, 0)),
              pl.BlockSpec((1, _NCLS), lambda i: (0, 0))],
    out_specs=pl.BlockSpec((_NG, _NCLS), lambda i: (0, 0)),
    out_shape=jax.ShapeDtypeStruct((_NG, _NCLS), jnp.float32),
    scratch_shapes=[pltpu.VMEM((_NG, _H), jnp.float32),
                    pltpu.VMEM((_NG, _H), jnp.float32)],
)


def kernel(x, edge_index, edge_attr, batch, node_W, node_b,
           lin1_W, lin1_b, lin2_W, lin3_W, lin3_b, pred_W, pred_b):
    src = edge_index[0]
    dst = edge_index[1]
    zrows = jnp.zeros((_RPS, _H), jnp.float32)
    zdeg = jnp.zeros((_N,), jnp.float32)

    a, b, c = _dense0(x, node_W, node_b.reshape(1, _H),
                      lin1_W[0], lin1_b[0].reshape(1, _H),
                      lin2_W[0], lin3_W[0], lin3_b[0].reshape(1, _H))
    part, degf = _edge_pass_deg(src, dst, edge_attr, a, zrows, zdeg)
    degc = degf.reshape(_NSC, _NB, _R, 1)

    for l in (1, 2):
        a, b, c = _combine_dense(part, part, degc, degc, b, c,
                                 lin1_W[l], lin1_b[l].reshape(1, _H),
                                 lin2_W[l], lin3_W[l],
                                 lin3_b[l].reshape(1, _H))
        (part,) = _edge_pass(src, dst, edge_attr, a, zrows)

    return _pool_head(part, part, degc, degc, b, c,
                      batch.reshape(_NB, 1, _R), pred_W,
                      pred_b.reshape(1, _NCLS))


# speedup vs baseline: 12.0611x; 2.3121x over previous
import functools

import jax
import jax.numpy as jnp
from jax import lax
from jax.experimental import pallas as pl
from jax.experimental.pallas import tpu as pltpu
from jax.experimental.pallas import tpu_sc as plsc

_N = 10000
_E = 320000
_H = 128
_NG = 128
_NCLS = 10
_NSC = 2
_NSUB = 16
_NW = _NSC * _NSUB
_EPW = _E // _NW
_K = 80
_NCH = _EPW // _K
_NSLOT = 4
_DK = 125
_DNCH = _EPW // _DK
_DSTG = 2000
_RPS = _N // _NSUB
_R = 1000
_NB = _N // _R

_mesh = plsc.VectorSubcoreMesh(
    core_axis_name="c", subcore_axis_name="s",
    num_cores=_NSC, num_subcores=_NSUB)


def _edge_body(src_hbm, dst_hbm, w_hbm, a_hbm, zrows_hbm,
               out_hbm, sring, dring, wring, rows_a, rows_b,
               acc_sh, gsem, sema, semb, isem0, isem1, isem2, isem3):
    cid = lax.axis_index("c")
    sid = lax.axis_index("s")
    wid = sid * _NSC + cid
    isems = (isem0, isem1, isem2, isem3)

    pltpu.sync_copy(zrows_hbm, acc_sh.at[pl.ds(sid * _RPS, _RPS)])
    plsc.subcore_barrier()

    ebase = wid * _NCH

    def istart(ch):
        slot = lax.rem(ch, _NSLOT)
        off = (ebase + ch) * _K
        for s in range(_NSLOT):
            @pl.when(slot == s)
            def _():
                pltpu.async_copy(src_hbm.at[pl.ds(off, _K)], sring.at[s],
                                 isems[s])
                pltpu.async_copy(dst_hbm.at[pl.ds(off, _K)], dring.at[s],
                                 isems[s])
                pltpu.async_copy(w_hbm.at[pl.ds(off, _K)], wring.at[s],
                                 isems[s])

    def iwait(ch):
        slot = lax.rem(ch, _NSLOT)
        for s in range(_NSLOT):
            @pl.when(slot == s)
            def _():
                pltpu.make_async_copy(src_hbm.at[pl.ds(0, _K)],
                                      sring.at[s], isems[s]).wait()
                pltpu.make_async_copy(dst_hbm.at[pl.ds(0, _K)],
                                      dring.at[s], isems[s]).wait()
                pltpu.make_async_copy(w_hbm.at[pl.ds(0, _K)],
                                      wring.at[s], isems[s]).wait()

    def gstart(ch, buf):
        slot = lax.rem(ch, _NSLOT)
        pltpu.async_copy(a_hbm.at[sring.at[slot]], buf, gsem)

    def gwait(buf):
        pltpu.make_async_copy(a_hbm.at[sring.at[0]], buf, gsem).wait()

    def sstart(ch, buf, sem):
        slot = lax.rem(ch, _NSLOT)
        pltpu.async_copy(buf, acc_sh.at[dring.at[slot]], sem, add=True)

    def swait(buf, sem):
        pltpu.make_async_copy(buf, acc_sh.at[dring.at[0]], sem).wait()

    def scale(ch, buf):
        slot = lax.rem(ch, _NSLOT)

        def grp(g, c2):
            w16 = wring[slot, pl.ds(g * 16, 16)]
            for j in range(16):
                wv = w16[j]
                e = g * 16 + j
                for k in range(_H // 16):
                    sl = pl.ds(k * 16, 16)
                    buf[e, sl] = buf[e, sl] * wv
            return c2
        lax.fori_loop(0, _K // 16, grp, 0)

    istart(0)
    istart(1)
    istart(2)
    iwait(0)
    gstart(0, rows_a)

    def pair(m, carry):
        ch = 2 * m
        gwait(rows_a)

        @pl.when(m > 0)
        def _wb():
            swait(rows_b, semb)

        @pl.when(ch + 1 < _NCH)
        def _gb():
            iwait(ch + 1)
            gstart(ch + 1, rows_b)

        @pl.when(ch + 3 < _NCH)
        def _pf0():
            istart(ch + 3)
        scale(ch, rows_a)
        sstart(ch, rows_a, sema)

        @pl.when(ch + 1 < _NCH)
        def _odd():
            gwait(rows_b)

            @pl.when(ch + 2 < _NCH)
            def _ga():
                swait(rows_a, sema)
                iwait(ch + 2)
                gstart(ch + 2, rows_a)

                @pl.when(ch + 4 < _NCH)
                def _pf1():
                    istart(ch + 4)
            scale(ch + 1, rows_b)
            sstart(ch + 1, rows_b, semb)
        return carry

    lax.fori_loop(0, (_NCH + 1) // 2, pair, 0)
    swait(rows_a, sema)
    plsc.subcore_barrier()

    @pl.when(sid < 2)
    def _():
        half = _N // 2
        pltpu.sync_copy(acc_sh.at[pl.ds(sid * half, half)],
                        out_hbm.at[pl.ds(cid * _N + sid * half, half)])


_edge_pass = pl.kernel(
    _edge_body,
    out_type=[jax.ShapeDtypeStruct((_NSC * _N, _H), jnp.float32)],
    mesh=_mesh,
    scratch_types=[
        pltpu.VMEM((_NSLOT, _K), jnp.int32),
        pltpu.VMEM((_NSLOT, _K), jnp.int32),
        pltpu.VMEM((_NSLOT, _K), jnp.float32),
        pltpu.VMEM((_K, _H), jnp.float32),
        pltpu.VMEM((_K, _H), jnp.float32),
        pltpu.VMEM_SHARED((_N, _H), jnp.float32),
    ] + [pltpu.SemaphoreType.DMA] * 7)


def _deg_body(dst_hbm, w_hbm, zdeg_hbm, deg_out_hbm,
              dst_v, w_v, stg_v, deg_sh, sem):
    cid = lax.axis_index("c")
    sid = lax.axis_index("s")
    wid = sid * _NSC + cid

    pltpu.sync_copy(dst_hbm.at[wid], dst_v)
    pltpu.sync_copy(w_hbm.at[wid], w_v)

    @pl.when(sid == 0)
    def _():
        def zchunk(j, c2):
            pltpu.sync_copy(zdeg_hbm.at[pl.ds(j * _DSTG, _DSTG)], stg_v)
            pltpu.sync_copy(stg_v, deg_sh.at[pl.ds(j * _DSTG, _DSTG)])
            return c2
        lax.fori_loop(0, _N // _DSTG, zchunk, 0)
    plsc.subcore_barrier()

    def group(g, c2):
        for j in range(8):
            ch = g * 8 + j
            pltpu.async_copy(w_v.at[ch], deg_sh.at[dst_v.at[ch]], sem,
                             add=True)
        for j in range(8):
            pltpu.make_async_copy(w_v.at[0], deg_sh.at[dst_v.at[0]],
                                  sem).wait()
        return c2
    lax.fori_loop(0, _DNCH // 8, group, 0)
    plsc.subcore_barrier()

    @pl.when(sid == 0)
    def _():
        def wchunk(j, c2):
            pltpu.sync_copy(deg_sh.at[pl.ds(j * _DSTG, _DSTG)], stg_v)
            pltpu.sync_copy(stg_v,
                            deg_out_hbm.at[pl.ds(cid * _N + j * _DSTG,
                                                 _DSTG)])
            return c2
        lax.fori_loop(0, _N // _DSTG, wchunk, 0)


_deg_pass = pl.kernel(
    _deg_body,
    out_type=[jax.ShapeDtypeStruct((_NSC * _N,), jnp.float32)],
    mesh=_mesh,
    scratch_types=[
        pltpu.VMEM((_DNCH, _DK), jnp.int32),
        pltpu.VMEM((_DNCH, _DK), jnp.float32),
        pltpu.VMEM((_DSTG,), jnp.float32),
        pltpu.VMEM_SHARED((_N,), jnp.float32),
        pltpu.SemaphoreType.DMA,
    ])


def _dot(a, b):
    return jnp.dot(a, b, preferred_element_type=jnp.float32)


def _dense0_body(x_ref, nw, nb, w1, b1, w2, w3, b3, a_ref, b_ref, c_ref):
    h = _dot(x_ref[...], nw[...]) + nb[...]
    a_ref[...] = _dot(h, w1[...]) + b1[...]
    b_ref[...] = _dot(h, w2[...])
    c_ref[...] = _dot(h, w3[...]) + b3[...]


def _combine_body(p0, p1, d0, d1, bp, cp, w1, b1, w2, w3, b3,
                  a_ref, b_ref, c_ref):
    deg = d0[0, 0] + d1[0, 0]
    h = jnp.maximum(p0[...] + p1[...] - deg * bp[...] + cp[...], 0.0)
    a_ref[...] = _dot(h, w1[...]) + b1[...]
    b_ref[...] = _dot(h, w2[...])
    c_ref[...] = _dot(h, w3[...]) + b3[...]


def _pool_body(p0, p1, d0, d1, bp, cp, batch_ref, pw, pb, out_ref,
               sums, cnt):
    i = pl.program_id(0)

    @pl.when(i == 0)
    def _():
        sums[...] = jnp.zeros_like(sums)
        cnt[...] = jnp.zeros_like(cnt)

    deg = d0[0, 0] + d1[0, 0]
    h = jnp.maximum(p0[...] + p1[...] - deg * bp[...] + cp[...], 0.0)
    brow = batch_ref[0]
    gids = lax.broadcasted_iota(jnp.int32, (_NG, _R), 0)
    onehot = (gids == brow).astype(jnp.float32)
    sums[...] += _dot(onehot, h)
    cnt[...] += _dot(onehot, jnp.ones((_R, _H), jnp.float32))

    @pl.when(i == _NB - 1)
    def _():
        hg = sums[...] / jnp.maximum(cnt[...], 1.0)
        out_ref[...] = _dot(hg, pw[...]) + pb[...]


_rowspec = pl.BlockSpec((_R, _H), lambda i: (i, 0))
_rowspec1 = pl.BlockSpec((_R, _H), lambda i: (i + _NB, 0))
_wspec = pl.BlockSpec((_H, _H), lambda i: (0, 0))
_bspec = pl.BlockSpec((1, _H), lambda i: (0, 0))
_d0spec = pl.BlockSpec((1, 1, _R, 1), lambda i: (0, i, 0, 0))
_d1spec = pl.BlockSpec((1, 1, _R, 1), lambda i: (1, i, 0, 0))

_dense0 = pl.pallas_call(
    _dense0_body,
    grid=(_NB,),
    in_specs=[_rowspec, _wspec, _bspec, _wspec, _bspec, _wspec, _wspec,
              _bspec],
    out_specs=[_rowspec, _rowspec, _rowspec],
    out_shape=[jax.ShapeDtypeStruct((_N, _H), jnp.float32)] * 3,
)

_combine_dense = pl.pallas_call(
    _combine_body,
    grid=(_NB,),
    in_specs=[_rowspec, _rowspec1, _d0spec, _d1spec, _rowspec, _rowspec,
              _wspec, _bspec, _wspec, _wspec, _bspec],
    out_specs=[_rowspec, _rowspec, _rowspec],
    out_shape=[jax.ShapeDtypeStruct((_N, _H), jnp.float32)] * 3,
)

_pool_head = pl.pallas_call(
    _pool_body,
    grid=(_NB,),
    in_specs=[_rowspec, _rowspec1, _d0spec, _d1spec, _rowspec, _rowspec,
              pl.BlockSpec((1, 1, _R), lambda i: (i, 0, 0)),
              pl.BlockSpec((_H, _NCLS), lambda i: (0, 0)),
              pl.BlockSpec((1, _NCLS), lambda i: (0, 0))],
    out_specs=pl.BlockSpec((_NG, _NCLS), lambda i: (0, 0)),
    out_shape=jax.ShapeDtypeStruct((_NG, _NCLS), jnp.float32),
    scratch_shapes=[pltpu.VMEM((_NG, _H), jnp.float32),
                    pltpu.VMEM((_NG, _H), jnp.float32)],
)


def kernel(x, edge_index, edge_attr, batch, node_W, node_b,
           lin1_W, lin1_b, lin2_W, lin3_W, lin3_b, pred_W, pred_b):
    src = edge_index[0]
    dst = edge_index[1]
    wp = edge_attr
    dstd = edge_index[1].reshape(_NW, _DNCH, _DK)
    wd = edge_attr.reshape(_NW, _DNCH, _DK)
    zrows = jnp.zeros((_RPS, _H), jnp.float32)
    zdeg = jnp.zeros((_N,), jnp.float32)

    (degf,) = _deg_pass(dstd, wd, zdeg)
    degc = degf.reshape(_NSC, _NB, _R, 1)
    a, b, c = _dense0(x, node_W, node_b.reshape(1, _H),
                      lin1_W[0], lin1_b[0].reshape(1, _H),
                      lin2_W[0], lin3_W[0], lin3_b[0].reshape(1, _H))
    (part,) = _edge_pass(src, dst, wp, a, zrows)

    for l in (1, 2):
        a, b, c = _combine_dense(part, part, degc, degc, b, c,
                                 lin1_W[l], lin1_b[l].reshape(1, _H),
                                 lin2_W[l], lin3_W[l],
                                 lin3_b[l].reshape(1, _H))
        (part,) = _edge_pass(src, dst, wp, a, zrows)

    return _pool_head(part, part, degc, degc, b, c,
                      batch.reshape(_NB, 1, _R), pred_W,
                      pred_b.reshape(1, _NCLS))


# speedup vs baseline: 13.2647x; 1.0998x over previous
import functools

import jax
import jax.numpy as jnp
from jax import lax
from jax.experimental import pallas as pl
from jax.experimental.pallas import tpu as pltpu
from jax.experimental.pallas import tpu_sc as plsc

_N = 10000
_E = 320000
_H = 128
_NG = 128
_NCLS = 10
_NSC = 2
_NSUB = 16
_NW = _NSC * _NSUB
_EPW = _E // _NW
_K = 112
_NCH = 92
_EPAD = _NW * _NCH * _K
_NSLOT = 4
_DK = 125
_DNCH = _EPW // _DK
_DSTG = 2000
_RPS = _N // _NSUB
_R = 1000
_NB = _N // _R

_mesh = plsc.VectorSubcoreMesh(
    core_axis_name="c", subcore_axis_name="s",
    num_cores=_NSC, num_subcores=_NSUB)


def _edge_body(src_hbm, dst_hbm, w_hbm, a_hbm, zrows_hbm,
               out_hbm, sring, dring, wring, rows_a, rows_b,
               acc_sh, gsem, sema, semb, isem0, isem1, isem2, isem3):
    cid = lax.axis_index("c")
    sid = lax.axis_index("s")
    wid = sid * _NSC + cid
    isems = (isem0, isem1, isem2, isem3)

    pltpu.sync_copy(zrows_hbm, acc_sh.at[pl.ds(sid * _RPS, _RPS)])
    plsc.subcore_barrier()

    ebase = wid * _NCH

    def istart(ch, s):
        off = (ebase + ch) * _K
        pltpu.async_copy(src_hbm.at[pl.ds(off, _K)], sring.at[s], isems[s])
        pltpu.async_copy(dst_hbm.at[pl.ds(off, _K)], dring.at[s], isems[s])
        pltpu.async_copy(w_hbm.at[pl.ds(off, _K)], wring.at[s], isems[s])

    def iwait(s):
        pltpu.make_async_copy(src_hbm.at[pl.ds(0, _K)],
                              sring.at[s], isems[s]).wait()
        pltpu.make_async_copy(dst_hbm.at[pl.ds(0, _K)],
                              dring.at[s], isems[s]).wait()
        pltpu.make_async_copy(w_hbm.at[pl.ds(0, _K)],
                              wring.at[s], isems[s]).wait()

    def gstart(s, buf):
        pltpu.async_copy(a_hbm.at[sring.at[s]], buf, gsem)

    def gwait(buf):
        pltpu.make_async_copy(a_hbm.at[sring.at[0]], buf, gsem).wait()

    def sstart(s, buf, sem):
        pltpu.async_copy(buf, acc_sh.at[dring.at[s]], sem, add=True)

    def swait(buf, sem):
        pltpu.make_async_copy(buf, acc_sh.at[dring.at[0]], sem).wait()

    def scale(s, buf):
        def grp(g, c2):
            w16 = wring[s, pl.ds(g * 16, 16)]
            for j in range(16):
                wv = w16[j]
                e = g * 16 + j
                for k in range(_H // 16):
                    sl = pl.ds(k * 16, 16)
                    buf[e, sl] = buf[e, sl] * wv
            return c2
        lax.fori_loop(0, _K // 16, grp, 0)

    istart(0, 0)
    istart(1, 1)
    istart(2, 2)
    iwait(0)
    gstart(0, rows_a)

    def quad(t, carry):
        base = 4 * t
        for s in range(4):
            ch = base + s
            X, sX = (rows_a, sema) if s % 2 == 0 else (rows_b, semb)
            Y, sY = (rows_b, semb) if s % 2 == 0 else (rows_a, sema)
            gwait(X)

            @pl.when(ch > 0)
            def _w():
                swait(Y, sY)

            @pl.when(ch + 1 < _NCH)
            def _g():
                iwait((s + 1) % 4)
                gstart((s + 1) % 4, Y)

            @pl.when(ch + 3 < _NCH)
            def _p():
                istart(ch + 3, (s + 3) % 4)
            scale(s, X)
            sstart(s, X, sX)
        return carry

    lax.fori_loop(0, _NCH // 4, quad, 0)
    swait(rows_b, semb)
    plsc.subcore_barrier()

    @pl.when(sid < 2)
    def _():
        half = _N // 2
        pltpu.sync_copy(acc_sh.at[pl.ds(sid * half, half)],
                        out_hbm.at[pl.ds(cid * _N + sid * half, half)])


_edge_pass = pl.kernel(
    _edge_body,
    out_type=[jax.ShapeDtypeStruct((_NSC * _N, _H), jnp.float32)],
    mesh=_mesh,
    scratch_types=[
        pltpu.VMEM((_NSLOT, _K), jnp.int32),
        pltpu.VMEM((_NSLOT, _K), jnp.int32),
        pltpu.VMEM((_NSLOT, _K), jnp.float32),
        pltpu.VMEM((_K, _H), jnp.float32),
        pltpu.VMEM((_K, _H), jnp.float32),
        pltpu.VMEM_SHARED((_N, _H), jnp.float32),
    ] + [pltpu.SemaphoreType.DMA] * 7)


def _deg_body(dst_hbm, w_hbm, zdeg_hbm, deg_out_hbm,
              dst_v, w_v, stg_v, deg_sh, sem):
    cid = lax.axis_index("c")
    sid = lax.axis_index("s")
    wid = sid * _NSC + cid

    pltpu.sync_copy(dst_hbm.at[wid], dst_v)
    pltpu.sync_copy(w_hbm.at[wid], w_v)

    @pl.when(sid == 0)
    def _():
        def zchunk(j, c2):
            pltpu.sync_copy(zdeg_hbm.at[pl.ds(j * _DSTG, _DSTG)], stg_v)
            pltpu.sync_copy(stg_v, deg_sh.at[pl.ds(j * _DSTG, _DSTG)])
            return c2
        lax.fori_loop(0, _N // _DSTG, zchunk, 0)
    plsc.subcore_barrier()

    def group(g, c2):
        for j in range(8):
            ch = g * 8 + j
            pltpu.async_copy(w_v.at[ch], deg_sh.at[dst_v.at[ch]], sem,
                             add=True)
        for j in range(8):
            pltpu.make_async_copy(w_v.at[0], deg_sh.at[dst_v.at[0]],
                                  sem).wait()
        return c2
    lax.fori_loop(0, _DNCH // 8, group, 0)
    plsc.subcore_barrier()

    @pl.when(sid == 0)
    def _():
        def wchunk(j, c2):
            pltpu.sync_copy(deg_sh.at[pl.ds(j * _DSTG, _DSTG)], stg_v)
            pltpu.sync_copy(stg_v,
                            deg_out_hbm.at[pl.ds(cid * _N + j * _DSTG,
                                                 _DSTG)])
            return c2
        lax.fori_loop(0, _N // _DSTG, wchunk, 0)


_deg_pass = pl.kernel(
    _deg_body,
    out_type=[jax.ShapeDtypeStruct((_NSC * _N,), jnp.float32)],
    mesh=_mesh,
    scratch_types=[
        pltpu.VMEM((_DNCH, _DK), jnp.int32),
        pltpu.VMEM((_DNCH, _DK), jnp.float32),
        pltpu.VMEM((_DSTG,), jnp.float32),
        pltpu.VMEM_SHARED((_N,), jnp.float32),
        pltpu.SemaphoreType.DMA,
    ])


def _dot(a, b):
    return jnp.dot(a, b, preferred_element_type=jnp.float32)


def _dense0_body(x_ref, nw, nb, w1, b1, w2, w3, b3, a_ref, b_ref, c_ref):
    h = _dot(x_ref[...], nw[...]) + nb[...]
    a_ref[...] = _dot(h, w1[...]) + b1[...]
    b_ref[...] = _dot(h, w2[...])
    c_ref[...] = _dot(h, w3[...]) + b3[...]


def _combine_body(p0, p1, d0, d1, bp, cp, w1, b1, w2, w3, b3,
                  a_ref, b_ref, c_ref):
    deg = d0[0, 0] + d1[0, 0]
    h = jnp.maximum(p0[...] + p1[...] - deg * bp[...] + cp[...], 0.0)
    a_ref[...] = _dot(h, w1[...]) + b1[...]
    b_ref[...] = _dot(h, w2[...])
    c_ref[...] = _dot(h, w3[...]) + b3[...]


def _pool_body(p0, p1, d0, d1, bp, cp, batch_ref, pw, pb, out_ref,
               sums, cnt):
    i = pl.program_id(0)

    @pl.when(i == 0)
    def _():
        sums[...] = jnp.zeros_like(sums)
        cnt[...] = jnp.zeros_like(cnt)

    deg = d0[0, 0] + d1[0, 0]
    h = jnp.maximum(p0[...] + p1[...] - deg * bp[...] + cp[...], 0.0)
    brow = batch_ref[0]
    gids = lax.broadcasted_iota(jnp.int32, (_NG, _R), 0)
    onehot = (gids == brow).astype(jnp.float32)
    sums[...] += _dot(onehot, h)
    cnt[...] += _dot(onehot, jnp.ones((_R, _H), jnp.float32))

    @pl.when(i == _NB - 1)
    def _():
        hg = sums[...] / jnp.maximum(cnt[...], 1.0)
        out_ref[...] = _dot(hg, pw[...]) + pb[...]


_rowspec = pl.BlockSpec((_R, _H), lambda i: (i, 0))
_rowspec1 = pl.BlockSpec((_R, _H), lambda i: (i + _NB, 0))
_wspec = pl.BlockSpec((_H, _H), lambda i: (0, 0))
_bspec = pl.BlockSpec((1, _H), lambda i: (0, 0))
_d0spec = pl.BlockSpec((1, 1, _R, 1), lambda i: (0, i, 0, 0))
_d1spec = pl.BlockSpec((1, 1, _R, 1), lambda i: (1, i, 0, 0))

_dense0 = pl.pallas_call(
    _dense0_body,
    grid=(_NB,),
    in_specs=[_rowspec, _wspec, _bspec, _wspec, _bspec, _wspec, _wspec,
              _bspec],
    out_specs=[_rowspec, _rowspec, _rowspec],
    out_shape=[jax.ShapeDtypeStruct((_N, _H), jnp.float32)] * 3,
)

_combine_dense = pl.pallas_call(
    _combine_body,
    grid=(_NB,),
    in_specs=[_rowspec, _rowspec1, _d0spec, _d1spec, _rowspec, _rowspec,
              _wspec, _bspec, _wspec, _wspec, _bspec],
    out_specs=[_rowspec, _rowspec, _rowspec],
    out_shape=[jax.ShapeDtypeStruct((_N, _H), jnp.float32)] * 3,
)

_pool_head = pl.pallas_call(
    _pool_body,
    grid=(_NB,),
    in_specs=[_rowspec, _rowspec1, _d0spec, _d1spec, _rowspec, _rowspec,
              pl.BlockSpec((1, 1, _R), lambda i: (i, 0, 0)),
              pl.BlockSpec((_H, _NCLS), lambda i: (0, 0)),
              pl.BlockSpec((1, _NCLS), lambda i: (0, 0))],
    out_specs=pl.BlockSpec((_NG, _NCLS), lambda i: (0, 0)),
    out_shape=jax.ShapeDtypeStruct((_NG, _NCLS), jnp.float32),
    scratch_shapes=[pltpu.VMEM((_NG, _H), jnp.float32),
                    pltpu.VMEM((_NG, _H), jnp.float32)],
)


def kernel(x, edge_index, edge_attr, batch, node_W, node_b,
           lin1_W, lin1_b, lin2_W, lin3_W, lin3_b, pred_W, pred_b):
    npad = _EPAD - _E
    fill = (jnp.arange(npad, dtype=jnp.int32) * 37) % _N
    src = jnp.concatenate([edge_index[0], fill])
    dst = jnp.concatenate([edge_index[1], fill])
    wp = jnp.concatenate([edge_attr, jnp.zeros((npad,), jnp.float32)])
    dstd = edge_index[1].reshape(_NW, _DNCH, _DK)
    wd = edge_attr.reshape(_NW, _DNCH, _DK)
    zrows = jnp.zeros((_RPS, _H), jnp.float32)
    zdeg = jnp.zeros((_N,), jnp.float32)

    (degf,) = _deg_pass(dstd, wd, zdeg)
    degc = degf.reshape(_NSC, _NB, _R, 1)
    a, b, c = _dense0(x, node_W, node_b.reshape(1, _H),
                      lin1_W[0], lin1_b[0].reshape(1, _H),
                      lin2_W[0], lin3_W[0], lin3_b[0].reshape(1, _H))
    (part,) = _edge_pass(src, dst, wp, a, zrows)

    for l in (1, 2):
        a, b, c = _combine_dense(part, part, degc, degc, b, c,
                                 lin1_W[l], lin1_b[l].reshape(1, _H),
                                 lin2_W[l], lin3_W[l],
                                 lin3_b[l].reshape(1, _H))
        (part,) = _edge_pass(src, dst, wp, a, zrows)

    return _pool_head(part, part, degc, degc, b, c,
                      batch.reshape(_NB, 1, _R), pred_W,
                      pred_b.reshape(1, _NCLS))


# speedup vs baseline: 13.2984x; 1.0025x over previous
import functools

import jax
import jax.numpy as jnp
import numpy as _np
from jax import lax
from jax.experimental import pallas as pl
from jax.experimental.pallas import tpu as pltpu
from jax.experimental.pallas import tpu_sc as plsc

_N = 10000
_E = 320000
_H = 128
_NG = 128
_NCLS = 10
_NSC = 2
_NSUB = 16
_NW = _NSC * _NSUB
_EPW = _E // _NW
_K = 112
_NCH = 92
_EPAD = _NW * _NCH * _K
_NSLOT = 4
_DK = 125
_DNCH = _EPW // _DK
_DSTG = 2000
_ZR = 2000
_R = 1000
_NB = _N // _R

_mesh = plsc.VectorSubcoreMesh(
    core_axis_name="c", subcore_axis_name="s",
    num_cores=_NSC, num_subcores=_NSUB)


def _edge_body(src_hbm, dst_hbm, w_hbm, a_hbm, zrows_hbm,
               out_hbm, sring, dring, wring, rows_a, rows_b,
               acc_sh, gsem, sema, semb, isem0, isem1, isem2, isem3):
    cid = lax.axis_index("c")
    sid = lax.axis_index("s")
    wid = sid * _NSC + cid
    isems = (isem0, isem1, isem2, isem3)

    @pl.when(sid < _N // _ZR)
    def _():
        pltpu.sync_copy(zrows_hbm, acc_sh.at[pl.ds(sid * _ZR, _ZR)])
    plsc.subcore_barrier()

    ebase = wid * _NCH

    def istart(ch, s):
        off = (ebase + ch) * _K
        pltpu.async_copy(src_hbm.at[pl.ds(off, _K)], sring.at[s], isems[s])
        pltpu.async_copy(dst_hbm.at[pl.ds(off, _K)], dring.at[s], isems[s])
        pltpu.async_copy(w_hbm.at[pl.ds(off, _K)], wring.at[s], isems[s])

    def iwait(s):
        pltpu.make_async_copy(src_hbm.at[pl.ds(0, _K)],
                              sring.at[s], isems[s]).wait()
        pltpu.make_async_copy(dst_hbm.at[pl.ds(0, _K)],
                              dring.at[s], isems[s]).wait()
        pltpu.make_async_copy(w_hbm.at[pl.ds(0, _K)],
                              wring.at[s], isems[s]).wait()

    def gstart(s, buf):
        pltpu.async_copy(a_hbm.at[sring.at[s]], buf, gsem)

    def gwait(buf):
        pltpu.make_async_copy(a_hbm.at[sring.at[0]], buf, gsem).wait()

    def sstart(s, buf, sem):
        pltpu.async_copy(buf, acc_sh.at[dring.at[s]], sem, add=True)

    def swait(buf, sem):
        pltpu.make_async_copy(buf, acc_sh.at[dring.at[0]], sem).wait()

    def scale(s, buf):
        def grp(g, c2):
            w16 = wring[s, pl.ds(g * 16, 16)]
            for j in range(16):
                wv = w16[j]
                e = g * 16 + j
                for k in range(_H // 16):
                    sl = pl.ds(k * 16, 16)
                    buf[e, sl] = buf[e, sl] * wv
            return c2
        lax.fori_loop(0, _K // 16, grp, 0)

    istart(0, 0)
    istart(1, 1)
    istart(2, 2)
    iwait(0)
    gstart(0, rows_a)

    def quad(t, carry):
        base = 4 * t
        for s in range(4):
            ch = base + s
            X, sX = (rows_a, sema) if s % 2 == 0 else (rows_b, semb)
            Y, sY = (rows_b, semb) if s % 2 == 0 else (rows_a, sema)
            gwait(X)

            @pl.when(ch > 0)
            def _w():
                swait(Y, sY)

            @pl.when(ch + 1 < _NCH)
            def _g():
                iwait((s + 1) % 4)
                gstart((s + 1) % 4, Y)

            @pl.when(ch + 3 < _NCH)
            def _p():
                istart(ch + 3, (s + 3) % 4)
            scale(s, X)
            sstart(s, X, sX)
        return carry

    lax.fori_loop(0, _NCH // 4, quad, 0)
    swait(rows_b, semb)
    plsc.subcore_barrier()

    @pl.when(sid < _N // _ZR)
    def _():
        pltpu.sync_copy(acc_sh.at[pl.ds(sid * _ZR, _ZR)],
                        out_hbm.at[pl.ds(cid * _N + sid * _ZR, _ZR)])


_edge_pass = pl.kernel(
    _edge_body,
    out_type=[jax.ShapeDtypeStruct((_NSC * _N, _H), jnp.float32)],
    mesh=_mesh,
    scratch_types=[
        pltpu.VMEM((_NSLOT, _K), jnp.int32),
        pltpu.VMEM((_NSLOT, _K), jnp.int32),
        pltpu.VMEM((_NSLOT, _K), jnp.float32),
        pltpu.VMEM((_K, _H), jnp.float32),
        pltpu.VMEM((_K, _H), jnp.float32),
        pltpu.VMEM_SHARED((_N, _H), jnp.float32),
    ] + [pltpu.SemaphoreType.DMA] * 7)


def _deg_body(dst_hbm, w_hbm, zdeg_hbm, deg_out_hbm,
              dst_v, w_v, stg_v, deg_sh, sem):
    cid = lax.axis_index("c")
    sid = lax.axis_index("s")
    wid = sid * _NSC + cid

    pltpu.sync_copy(dst_hbm.at[wid], dst_v)
    pltpu.sync_copy(w_hbm.at[wid], w_v)

    @pl.when(sid == 0)
    def _():
        def zchunk(j, c2):
            pltpu.sync_copy(zdeg_hbm.at[pl.ds(j * _DSTG, _DSTG)], stg_v)
            pltpu.sync_copy(stg_v, deg_sh.at[pl.ds(j * _DSTG, _DSTG)])
            return c2
        lax.fori_loop(0, _N // _DSTG, zchunk, 0)
    plsc.subcore_barrier()

    def group(g, c2):
        for j in range(8):
            ch = g * 8 + j
            pltpu.async_copy(w_v.at[ch], deg_sh.at[dst_v.at[ch]], sem,
                             add=True)
        for j in range(8):
            pltpu.make_async_copy(w_v.at[0], deg_sh.at[dst_v.at[0]],
                                  sem).wait()
        return c2
    lax.fori_loop(0, _DNCH // 8, group, 0)
    plsc.subcore_barrier()

    @pl.when(sid == 0)
    def _():
        def wchunk(j, c2):
            pltpu.sync_copy(deg_sh.at[pl.ds(j * _DSTG, _DSTG)], stg_v)
            pltpu.sync_copy(stg_v,
                            deg_out_hbm.at[pl.ds(cid * _N + j * _DSTG,
                                                 _DSTG)])
            return c2
        lax.fori_loop(0, _N // _DSTG, wchunk, 0)


_deg_pass = pl.kernel(
    _deg_body,
    out_type=[jax.ShapeDtypeStruct((_NSC * _N,), jnp.float32)],
    mesh=_mesh,
    scratch_types=[
        pltpu.VMEM((_DNCH, _DK), jnp.int32),
        pltpu.VMEM((_DNCH, _DK), jnp.float32),
        pltpu.VMEM((_DSTG,), jnp.float32),
        pltpu.VMEM_SHARED((_N,), jnp.float32),
        pltpu.SemaphoreType.DMA,
    ])


def _dot(a, b):
    return jnp.dot(a, b, preferred_element_type=jnp.float32)


def _dense0_body(x_ref, nw, nb, w1, b1, w2, w3, b3, a_ref, b_ref, c_ref):
    h = _dot(x_ref[...], nw[...]) + nb[...]
    a_ref[...] = _dot(h, w1[...]) + b1[...]
    b_ref[...] = _dot(h, w2[...])
    c_ref[...] = _dot(h, w3[...]) + b3[...]


def _combine_body(p0, p1, d0, d1, bp, cp, w1, b1, w2, w3, b3,
                  a_ref, b_ref, c_ref):
    deg = d0[0, 0] + d1[0, 0]
    h = jnp.maximum(p0[...] + p1[...] - deg * bp[...] + cp[...], 0.0)
    a_ref[...] = _dot(h, w1[...]) + b1[...]
    b_ref[...] = _dot(h, w2[...])
    c_ref[...] = _dot(h, w3[...]) + b3[...]


def _pool_body(p0, p1, d0, d1, bp, cp, batch_ref, pw, pb, out_ref,
               sums, cnt):
    i = pl.program_id(0)

    @pl.when(i == 0)
    def _():
        sums[...] = jnp.zeros_like(sums)
        cnt[...] = jnp.zeros_like(cnt)

    deg = d0[0, 0] + d1[0, 0]
    h = jnp.maximum(p0[...] + p1[...] - deg * bp[...] + cp[...], 0.0)
    brow = batch_ref[0]
    gids = lax.broadcasted_iota(jnp.int32, (_NG, _R), 0)
    onehot = (gids == brow).astype(jnp.float32)
    sums[...] += _dot(onehot, h)
    cnt[...] += _dot(onehot, jnp.ones((_R, _H), jnp.float32))

    @pl.when(i == _NB - 1)
    def _():
        hg = sums[...] / jnp.maximum(cnt[...], 1.0)
        out_ref[...] = _dot(hg, pw[...]) + pb[...]


_rowspec = pl.BlockSpec((_R, _H), lambda i: (i, 0))
_rowspec1 = pl.BlockSpec((_R, _H), lambda i: (i + _NB, 0))
_wspec = pl.BlockSpec((_H, _H), lambda i: (0, 0))
_bspec = pl.BlockSpec((1, _H), lambda i: (0, 0))
_d0spec = pl.BlockSpec((1, 1, _R, 1), lambda i: (0, i, 0, 0))
_d1spec = pl.BlockSpec((1, 1, _R, 1), lambda i: (1, i, 0, 0))

_dense0 = pl.pallas_call(
    _dense0_body,
    grid=(_NB,),
    in_specs=[_rowspec, _wspec, _bspec, _wspec, _bspec, _wspec, _wspec,
              _bspec],
    out_specs=[_rowspec, _rowspec, _rowspec],
    out_shape=[jax.ShapeDtypeStruct((_N, _H), jnp.float32)] * 3,
)

_combine_dense = pl.pallas_call(
    _combine_body,
    grid=(_NB,),
    in_specs=[_rowspec, _rowspec1, _d0spec, _d1spec, _rowspec, _rowspec,
              _wspec, _bspec, _wspec, _wspec, _bspec],
    out_specs=[_rowspec, _rowspec, _rowspec],
    out_shape=[jax.ShapeDtypeStruct((_N, _H), jnp.float32)] * 3,
)

_pool_head = pl.pallas_call(
    _pool_body,
    grid=(_NB,),
    in_specs=[_rowspec, _rowspec1, _d0spec, _d1spec, _rowspec, _rowspec,
              pl.BlockSpec((1, 1, _R), lambda i: (i, 0, 0)),
              pl.BlockSpec((_H, _NCLS), lambda i: (0, 0)),
              pl.BlockSpec((1, _NCLS), lambda i: (0, 0))],
    out_specs=pl.BlockSpec((_NG, _NCLS), lambda i: (0, 0)),
    out_shape=jax.ShapeDtypeStruct((_NG, _NCLS), jnp.float32),
    scratch_shapes=[pltpu.VMEM((_NG, _H), jnp.float32),
                    pltpu.VMEM((_NG, _H), jnp.float32)],
)


def kernel(x, edge_index, edge_attr, batch, node_W, node_b,
           lin1_W, lin1_b, lin2_W, lin3_W, lin3_b, pred_W, pred_b):
    npad = _EPAD - _E
    fill = (jnp.arange(npad, dtype=jnp.int32) * 37) % _N
    src = jnp.concatenate([edge_index[0], fill])
    dst = jnp.concatenate([edge_index[1], fill])
    wp = jnp.concatenate([edge_attr, jnp.zeros((npad,), jnp.float32)])
    dstd = edge_index[1].reshape(_NW, _DNCH, _DK)
    wd = edge_attr.reshape(_NW, _DNCH, _DK)
    zrows = jnp.zeros((_ZR, _H), jnp.float32)
    zdeg = jnp.zeros((_N,), jnp.float32)

    (degf,) = _deg_pass(dstd, wd, zdeg)
    degc = degf.reshape(_NSC, _NB, _R, 1)
    a, b, c = _dense0(x, node_W, node_b.reshape(1, _H),
                      lin1_W[0], lin1_b[0].reshape(1, _H),
                      lin2_W[0], lin3_W[0], lin3_b[0].reshape(1, _H))
    (part,) = _edge_pass(src, dst, wp, a, zrows)

    for l in (1, 2):
        a, b, c = _combine_dense(part, part, degc, degc, b, c,
                                 lin1_W[l], lin1_b[l].reshape(1, _H),
                                 lin2_W[l], lin3_W[l],
                                 lin3_b[l].reshape(1, _H))
        (part,) = _edge_pass(src, dst, wp, a, zrows)

    return _pool_head(part, part, degc, degc, b, c,
                      batch.reshape(_NB, 1, _R), pred_W,
                      pred_b.reshape(1, _NCLS))


# speedup vs baseline: 13.6371x; 1.0255x over previous
import functools

import jax
import jax.numpy as jnp
from jax import lax
from jax.experimental import pallas as pl
from jax.experimental.pallas import tpu as pltpu
from jax.experimental.pallas import tpu_sc as plsc

_N = 10000
_E = 320000
_H = 128
_NG = 128
_NCLS = 10
_NSC = 2
_NSUB = 16
_NW = _NSC * _NSUB
_EPW = _E // _NW
_K = 128
_NCH = 80
_EPAD = _NW * _NCH * _K
_NSLOT = 4
_DK = 125
_DNCH = _EPW // _DK
_DSTG = 2000
_ZR = 2000
_R = 1000
_NB = _N // _R

_mesh = plsc.VectorSubcoreMesh(
    core_axis_name="c", subcore_axis_name="s",
    num_cores=_NSC, num_subcores=_NSUB)


def _edge_body(src_hbm, dst_hbm, w_hbm, a_hbm, zrows_hbm,
               out_hbm, sring, dring, wring, rows_a, rows_b,
               acc_sh, gsem, sema, semb, isem0, isem1, isem2, isem3):
    cid = lax.axis_index("c")
    sid = lax.axis_index("s")
    wid = sid * _NSC + cid
    isems = (isem0, isem1, isem2, isem3)

    @pl.when(sid < _N // _ZR)
    def _():
        pltpu.sync_copy(zrows_hbm, acc_sh.at[pl.ds(sid * _ZR, _ZR)])
    plsc.subcore_barrier()

    ebase = wid * _NCH

    def istart(ch, s):
        off = (ebase + ch) * _K
        pltpu.async_copy(src_hbm.at[pl.ds(off, _K)], sring.at[s], isems[s])
        pltpu.async_copy(dst_hbm.at[pl.ds(off, _K)], dring.at[s], isems[s])
        pltpu.async_copy(w_hbm.at[pl.ds(off, _K)], wring.at[s], isems[s])

    def iwait(s):
        pltpu.make_async_copy(src_hbm.at[pl.ds(0, _K)],
                              sring.at[s], isems[s]).wait()
        pltpu.make_async_copy(dst_hbm.at[pl.ds(0, _K)],
                              dring.at[s], isems[s]).wait()
        pltpu.make_async_copy(w_hbm.at[pl.ds(0, _K)],
                              wring.at[s], isems[s]).wait()

    def gstart(s, buf):
        pltpu.async_copy(a_hbm.at[sring.at[s]], buf, gsem)

    def gwait(buf):
        pltpu.make_async_copy(a_hbm.at[sring.at[0]], buf, gsem).wait()

    def sstart(s, buf, sem):
        pltpu.async_copy(buf, acc_sh.at[dring.at[s]], sem, add=True)

    def swait(buf, sem):
        pltpu.make_async_copy(buf, acc_sh.at[dring.at[0]], sem).wait()

    def scale(s, buf):
        def grp(g, c2):
            w16 = wring[s, pl.ds(g * 16, 16)]
            for j in range(16):
                wv = w16[j]
                e = g * 16 + j
                for k in range(_H // 16):
                    sl = pl.ds(k * 16, 16)
                    buf[e, sl] = buf[e, sl] * wv
            return c2
        lax.fori_loop(0, _K // 16, grp, 0)

    istart(0, 0)
    istart(1, 1)
    istart(2, 2)
    iwait(0)
    gstart(0, rows_a)

    def quad(t, carry):
        base = 4 * t
        for s in range(4):
            ch = base + s
            X, sX = (rows_a, sema) if s % 2 == 0 else (rows_b, semb)
            Y, sY = (rows_b, semb) if s % 2 == 0 else (rows_a, sema)
            gwait(X)

            @pl.when(ch > 0)
            def _w():
                swait(Y, sY)

            @pl.when(ch + 1 < _NCH)
            def _g():
                iwait((s + 1) % 4)
                gstart((s + 1) % 4, Y)

            @pl.when(ch + 3 < _NCH)
            def _p():
                istart(ch + 3, (s + 3) % 4)
            scale(s, X)
            sstart(s, X, sX)
        return carry

    lax.fori_loop(0, _NCH // 4, quad, 0)
    swait(rows_b, semb)
    plsc.subcore_barrier()

    @pl.when(sid < _N // _ZR)
    def _():
        pltpu.sync_copy(acc_sh.at[pl.ds(sid * _ZR, _ZR)],
                        out_hbm.at[pl.ds(cid * _N + sid * _ZR, _ZR)])


_edge_pass = pl.kernel(
    _edge_body,
    out_type=[jax.ShapeDtypeStruct((_NSC * _N, _H), jnp.float32)],
    mesh=_mesh,
    scratch_types=[
        pltpu.VMEM((_NSLOT, _K), jnp.int32),
        pltpu.VMEM((_NSLOT, _K), jnp.int32),
        pltpu.VMEM((_NSLOT, _K), jnp.float32),
        pltpu.VMEM((_K, _H), jnp.float32),
        pltpu.VMEM((_K, _H), jnp.float32),
        pltpu.VMEM_SHARED((_N, _H), jnp.float32),
    ] + [pltpu.SemaphoreType.DMA] * 7)


def _deg_body(dst_hbm, w_hbm, zdeg_hbm, deg_out_hbm,
              dst_v, w_v, stg_v, deg_sh, sem):
    cid = lax.axis_index("c")
    sid = lax.axis_index("s")
    wid = sid * _NSC + cid

    pltpu.sync_copy(dst_hbm.at[wid], dst_v)
    pltpu.sync_copy(w_hbm.at[wid], w_v)

    @pl.when(sid == 0)
    def _():
        def zchunk(j, c2):
            pltpu.sync_copy(zdeg_hbm.at[pl.ds(j * _DSTG, _DSTG)], stg_v)
            pltpu.sync_copy(stg_v, deg_sh.at[pl.ds(j * _DSTG, _DSTG)])
            return c2
        lax.fori_loop(0, _N // _DSTG, zchunk, 0)
    plsc.subcore_barrier()

    def group(g, c2):
        for j in range(8):
            ch = g * 8 + j
            pltpu.async_copy(w_v.at[ch], deg_sh.at[dst_v.at[ch]], sem,
                             add=True)
        for j in range(8):
            pltpu.make_async_copy(w_v.at[0], deg_sh.at[dst_v.at[0]],
                                  sem).wait()
        return c2
    lax.fori_loop(0, _DNCH // 8, group, 0)
    plsc.subcore_barrier()

    @pl.when(sid == 0)
    def _():
        def wchunk(j, c2):
            pltpu.sync_copy(deg_sh.at[pl.ds(j * _DSTG, _DSTG)], stg_v)
            pltpu.sync_copy(stg_v,
                            deg_out_hbm.at[pl.ds(cid * _N + j * _DSTG,
                                                 _DSTG)])
            return c2
        lax.fori_loop(0, _N // _DSTG, wchunk, 0)


_deg_pass = pl.kernel(
    _deg_body,
    out_type=[jax.ShapeDtypeStruct((_NSC * _N,), jnp.float32)],
    mesh=_mesh,
    scratch_types=[
        pltpu.VMEM((_DNCH, _DK), jnp.int32),
        pltpu.VMEM((_DNCH, _DK), jnp.float32),
        pltpu.VMEM((_DSTG,), jnp.float32),
        pltpu.VMEM_SHARED((_N,), jnp.float32),
        pltpu.SemaphoreType.DMA,
    ])


def _dot(a, b):
    return jnp.dot(a, b, preferred_element_type=jnp.float32)


def _dense0_body(x_ref, nw, nb, w1, b1, w2, w3, b3, a_ref, b_ref, c_ref):
    h = _dot(x_ref[...], nw[...]) + nb[...]
    a_ref[...] = _dot(h, w1[...]) + b1[...]
    b_ref[...] = _dot(h, w2[...])
    c_ref[...] = _dot(h, w3[...]) + b3[...]


def _combine_body(p0, p1, d0, d1, bp, cp, w1, b1, w2, w3, b3,
                  a_ref, b_ref, c_ref):
    deg = d0[0, 0] + d1[0, 0]
    h = jnp.maximum(p0[...] + p1[...] - deg * bp[...] + cp[...], 0.0)
    a_ref[...] = _dot(h, w1[...]) + b1[...]
    b_ref[...] = _dot(h, w2[...])
    c_ref[...] = _dot(h, w3[...]) + b3[...]


def _pool_body(p0, p1, d0, d1, bp, cp, batch_ref, pw, pb, out_ref,
               sums, cnt):
    i = pl.program_id(0)

    @pl.when(i == 0)
    def _():
        sums[...] = jnp.zeros_like(sums)
        cnt[...] = jnp.zeros_like(cnt)

    deg = d0[0, 0] + d1[0, 0]
    h = jnp.maximum(p0[...] + p1[...] - deg * bp[...] + cp[...], 0.0)
    brow = batch_ref[0]
    gids = lax.broadcasted_iota(jnp.int32, (_NG, _R), 0)
    onehot = (gids == brow).astype(jnp.float32)
    sums[...] += _dot(onehot, h)
    cnt[...] += _dot(onehot, jnp.ones((_R, _H), jnp.float32))

    @pl.when(i == _NB - 1)
    def _():
        hg = sums[...] / jnp.maximum(cnt[...], 1.0)
        out_ref[...] = _dot(hg, pw[...]) + pb[...]


_rowspec = pl.BlockSpec((_R, _H), lambda i: (i, 0))
_rowspec1 = pl.BlockSpec((_R, _H), lambda i: (i + _NB, 0))
_wspec = pl.BlockSpec((_H, _H), lambda i: (0, 0))
_bspec = pl.BlockSpec((1, _H), lambda i: (0, 0))
_d0spec = pl.BlockSpec((1, 1, _R, 1), lambda i: (0, i, 0, 0))
_d1spec = pl.BlockSpec((1, 1, _R, 1), lambda i: (1, i, 0, 0))

_dense0 = pl.pallas_call(
    _dense0_body,
    grid=(_NB,),
    in_specs=[_rowspec, _wspec, _bspec, _wspec, _bspec, _wspec, _wspec,
              _bspec],
    out_specs=[_rowspec, _rowspec, _rowspec],
    out_shape=[jax.ShapeDtypeStruct((_N, _H), jnp.float32)] * 3,
)

_combine_dense = pl.pallas_call(
    _combine_body,
    grid=(_NB,),
    in_specs=[_rowspec, _rowspec1, _d0spec, _d1spec, _rowspec, _rowspec,
              _wspec, _bspec, _wspec, _wspec, _bspec],
    out_specs=[_rowspec, _rowspec, _rowspec],
    out_shape=[jax.ShapeDtypeStruct((_N, _H), jnp.float32)] * 3,
)

_pool_head = pl.pallas_call(
    _pool_body,
    grid=(_NB,),
    in_specs=[_rowspec, _rowspec1, _d0spec, _d1spec, _rowspec, _rowspec,
              pl.BlockSpec((1, 1, _R), lambda i: (i, 0, 0)),
              pl.BlockSpec((_H, _NCLS), lambda i: (0, 0)),
              pl.BlockSpec((1, _NCLS), lambda i: (0, 0))],
    out_specs=pl.BlockSpec((_NG, _NCLS), lambda i: (0, 0)),
    out_shape=jax.ShapeDtypeStruct((_NG, _NCLS), jnp.float32),
    scratch_shapes=[pltpu.VMEM((_NG, _H), jnp.float32),
                    pltpu.VMEM((_NG, _H), jnp.float32)],
)


def kernel(x, edge_index, edge_attr, batch, node_W, node_b,
           lin1_W, lin1_b, lin2_W, lin3_W, lin3_b, pred_W, pred_b):
    npad = _EPAD - _E
    fill = (jnp.arange(npad, dtype=jnp.int32) * 37) % _N
    src = jnp.concatenate([edge_index[0], fill])
    dst = jnp.concatenate([edge_index[1], fill])
    wp = jnp.concatenate([edge_attr, jnp.zeros((npad,), jnp.float32)])
    dstd = edge_index[1].reshape(_NW, _DNCH, _DK)
    wd = edge_attr.reshape(_NW, _DNCH, _DK)
    zrows = jnp.zeros((_ZR, _H), jnp.float32)
    zdeg = jnp.zeros((_N,), jnp.float32)

    (degf,) = _deg_pass(dstd, wd, zdeg)
    degc = degf.reshape(_NSC, _NB, _R, 1)
    a, b, c = _dense0(x, node_W, node_b.reshape(1, _H),
                      lin1_W[0], lin1_b[0].reshape(1, _H),
                      lin2_W[0], lin3_W[0], lin3_b[0].reshape(1, _H))
    (part,) = _edge_pass(src, dst, wp, a, zrows)

    for l in (1, 2):
        a, b, c = _combine_dense(part, part, degc, degc, b, c,
                                 lin1_W[l], lin1_b[l].reshape(1, _H),
                                 lin2_W[l], lin3_W[l],
                                 lin3_b[l].reshape(1, _H))
        (part,) = _edge_pass(src, dst, wp, a, zrows)

    return _pool_head(part, part, degc, degc, b, c,
                      batch.reshape(_NB, 1, _R), pred_W,
                      pred_b.reshape(1, _NCLS))
